# baseline (speedup 1.0000x reference)
import jax
import jax.numpy as jnp
from jax.experimental import pallas as pl
from jax.experimental.pallas import tpu as pltpu


def _chamfer_kernel(x_ref, yt_ref, ipc_ref, opr_ref, acc_ref):
    i = pl.program_id(0)

    x = x_ref[0]
    yt = yt_ref[0]
    in_col = ipc_ref[0] != 0
    out_row = opr_ref[0] != 0

    f32 = jnp.float32
    x2 = jnp.sum(x * x, axis=1, keepdims=True)
    y2 = jnp.sum(yt * yt, axis=0, keepdims=True)
    xy = jnp.dot(x, yt, preferred_element_type=f32)
    d2 = jnp.maximum(x2 + y2 - 2.0 * xy, 0.0)

    inf = jnp.inf
    min_xy = jnp.sqrt(jnp.min(jnp.where(out_row, d2, inf), axis=1,
                              keepdims=True))
    min_yx = jnp.sqrt(jnp.min(jnp.where(in_col, d2, inf), axis=0,
                              keepdims=True))

    in_f = in_col.astype(f32)
    out_f = out_row.astype(f32)
    cnt_in = jnp.sum(in_f)
    cnt_out = jnp.sum(out_f)
    n_in = jnp.maximum(1.0, cnt_in)
    n_out = jnp.maximum(1.0, cnt_out)

    sum_xy = jnp.sum(jnp.where(in_col, min_xy, 0.0))
    sum_yx = jnp.sum(jnp.where(out_row, min_yx, 0.0))
    e_both = 0.5 * (sum_xy / n_out + sum_yx / n_in)

    x_norm_sum = jnp.sum(in_f * jnp.sqrt(x2))
    e_nz = jnp.where(cnt_out == 0.0, x_norm_sum / n_in,
                     jnp.where(cnt_in == 0.0, x_norm_sum / n_out, e_both))

    n_oz = jnp.maximum(1.0, 256.0 - cnt_out)
    e_z = jnp.sum((1.0 - out_f) * jnp.sqrt(y2)) / n_oz

    lane = jax.lax.broadcasted_iota(jnp.int32, (1, 128), 1)
    vec = jnp.where(lane == 0, e_nz, jnp.where(lane == 1, e_z, 0.0))

    @pl.when(i == 0)
    def _init():
        acc_ref[...] = jnp.zeros_like(acc_ref)

    acc_ref[...] += vec


def kernel(target, reco, in_pid, out_pid):
    n_batches = target.shape[0]
    yt = reco.transpose(0, 2, 1)
    ipc = in_pid.reshape(n_batches, 256, 1)
    opr = out_pid.reshape(n_batches, 1, 256)

    acc = pl.pallas_call(
        _chamfer_kernel,
        grid=(n_batches,),
        in_specs=[
            pl.BlockSpec((1, 256, 4), lambda i: (i, 0, 0)),
            pl.BlockSpec((1, 4, 256), lambda i: (i, 0, 0)),
            pl.BlockSpec((1, 256, 1), lambda i: (i, 0, 0)),
            pl.BlockSpec((1, 1, 256), lambda i: (i, 0, 0)),
        ],
        out_specs=pl.BlockSpec((1, 128), lambda i: (0, 0)),
        out_shape=jax.ShapeDtypeStruct((1, 128), jnp.float32),
        compiler_params=pltpu.CompilerParams(
            dimension_semantics=("arbitrary",)),
    )(target, yt, ipc, opr)

    inv = 1.0 / n_batches
    return acc[0, 0] * inv, acc[0, 1] * inv


# speedup vs baseline: 1.8469x; 1.8469x over previous
import jax
import jax.numpy as jnp
from jax.experimental import pallas as pl
from jax.experimental.pallas import tpu as pltpu

_E = 8


def _chamfer_kernel(x_ref, yt_ref, ipc_ref, opr_ref, out_ref):
    x = x_ref[...]
    yt = yt_ref[...]
    in_col = ipc_ref[...] != 0
    out_row = opr_ref[...] != 0

    f32 = jnp.float32
    x2 = jnp.sum(x * x, axis=2, keepdims=True)
    y2 = jnp.sum(yt * yt, axis=1, keepdims=True)
    xy = jax.lax.dot_general(
        x, yt, dimension_numbers=(((2,), (1,)), ((0,), (0,))),
        preferred_element_type=f32)
    d2 = jnp.maximum(x2 + y2 - 2.0 * xy, 0.0)

    inf = jnp.inf
    min_xy = jnp.sqrt(jnp.min(jnp.where(out_row, d2, inf), axis=2,
                              keepdims=True))
    min_yx = jnp.sqrt(jnp.min(jnp.where(in_col, d2, inf), axis=1,
                              keepdims=True))

    in_f = in_col.astype(f32)
    out_f = out_row.astype(f32)
    cnt_in = jnp.sum(in_f, axis=(1, 2), keepdims=True)
    cnt_out = jnp.sum(out_f, axis=(1, 2), keepdims=True)
    n_in = jnp.maximum(1.0, cnt_in)
    n_out = jnp.maximum(1.0, cnt_out)

    sum_xy = jnp.sum(in_f * min_xy, axis=(1, 2), keepdims=True)
    sum_yx = jnp.sum(out_f * min_yx, axis=(1, 2), keepdims=True)
    e_both = 0.5 * (sum_xy / n_out + sum_yx / n_in)

    x_norm_sum = jnp.sum(in_f * jnp.sqrt(x2), axis=(1, 2), keepdims=True)
    e_nz = jnp.where(cnt_out == 0.0, x_norm_sum / n_in,
                     jnp.where(cnt_in == 0.0, x_norm_sum / n_out, e_both))

    n_oz = jnp.maximum(1.0, 256.0 - cnt_out)
    e_z = jnp.sum((1.0 - out_f) * jnp.sqrt(y2), axis=(1, 2),
                  keepdims=True) / n_oz

    s_nz = jnp.sum(e_nz)
    s_z = jnp.sum(e_z)
    lane = jax.lax.broadcasted_iota(jnp.int32, (1, 1, 128), 2)
    out_ref[...] = jnp.where(lane == 0, s_nz, jnp.where(lane == 1, s_z, 0.0))


def kernel(target, reco, in_pid, out_pid):
    n_batches = target.shape[0]
    n_steps = n_batches // _E
    yt = reco.transpose(0, 2, 1)
    ipc = in_pid.reshape(n_batches, 256, 1)
    opr = out_pid.reshape(n_batches, 1, 256)

    part = pl.pallas_call(
        _chamfer_kernel,
        grid=(n_steps,),
        in_specs=[
            pl.BlockSpec((_E, 256, 4), lambda i: (i, 0, 0)),
            pl.BlockSpec((_E, 4, 256), lambda i: (i, 0, 0)),
            pl.BlockSpec((_E, 256, 1), lambda i: (i, 0, 0)),
            pl.BlockSpec((_E, 1, 256), lambda i: (i, 0, 0)),
        ],
        out_specs=pl.BlockSpec((1, 1, 128), lambda i: (i, 0, 0)),
        out_shape=jax.ShapeDtypeStruct((n_steps, 1, 128), jnp.float32),
        compiler_params=pltpu.CompilerParams(
            dimension_semantics=("parallel",)),
    )(target, yt, ipc, opr)

    inv = 1.0 / n_batches
    return jnp.sum(part[:, 0, 0]) * inv, jnp.sum(part[:, 0, 1]) * inv
